# baseline (speedup 1.0000x reference)
import functools

import jax
import jax.numpy as jnp
from jax.experimental import pallas as pl

_RES = (16, 20, 24, 28, 32, 36, 40, 44)
_B, _CIN, _COUT = 2, 8, 8
_T = 4096


def _offsets():
    o = [0]
    for r in _RES:
        o.append(o[-1] + r ** 3)
    return tuple(o)


_OFF = _offsets()


def _round_up(x, m):
    return (x + m - 1) // m * m


def _lvl_body(x_ref, w_ref, b_ref, o_ref, *, R, h, T):
    t = pl.program_id(1)
    pieces = []
    for dx in range(3):
        for dy in range(3):
            for dz in range(3):
                off = (dx - 1) * R * R + (dy - 1) * R + (dz - 1)
                start = h + off
                a = start // 128 * 128
                rem = start - a
                if rem == 0:
                    sl = x_ref[0, :, pl.ds(t * T + a, T)]
                else:
                    raw = x_ref[0, :, pl.ds(t * T + a, T + 128)]
                    sl = jax.lax.slice(raw, (0, rem), (_CIN, rem + T))
                pieces.append(sl)
    feats = jnp.concatenate(pieces, axis=0)
    acc = jnp.dot(w_ref[...], feats,
                  preferred_element_type=jnp.float32)
    o_ref[0] = acc + b_ref[...]


def _lvl_conv(xp, wl, bl, *, R, sp, h, nt, spad, interpret=False):
    body = functools.partial(_lvl_body, R=R, h=h, T=_T)
    return pl.pallas_call(
        body,
        grid=(_B, nt),
        in_specs=[
            pl.BlockSpec((1, _CIN, spad), lambda b, t: (b, 0, 0)),
            pl.BlockSpec((_COUT, 27 * _CIN), lambda b, t: (0, 0)),
            pl.BlockSpec((_COUT, 1), lambda b, t: (0, 0)),
        ],
        out_specs=pl.BlockSpec((1, _COUT, _T), lambda b, t: (b, 0, t)),
        out_shape=jax.ShapeDtypeStruct((_B, _COUT, sp), jnp.float32),
        interpret=interpret,
    )(xp, wl, bl)


def kernel(input, offsets, resolutions, weight, bias):
    xt = jnp.transpose(input, (0, 2, 1))
    outs = []
    for l, r in enumerate(_RES):
        s = r ** 3
        R = r + 2
        sp = R ** 3
        h = R * R + R + 1
        nt = -(-sp // _T)
        spad = (nt - 1) * _T + _round_up(_T + 2 * h, 128) + 128
        seg = jax.lax.slice_in_dim(xt, _OFF[l], _OFF[l] + s, axis=2)
        seg3 = seg.reshape(_B, _CIN, r, r, r)
        segp = jnp.pad(seg3, ((0, 0), (0, 0), (1, 1), (1, 1), (1, 1)))
        flat = segp.reshape(_B, _CIN, sp)
        xp = jnp.pad(flat, ((0, 0), (0, 0), (h, spad - sp - h)))
        wl = jnp.transpose(weight[l].reshape(27 * _CIN, _COUT))
        bl = bias[l].reshape(_COUT, 1)
        o = _lvl_conv(xp, wl, bl, R=R, sp=sp, h=h, nt=nt, spad=spad)
        outs.append(o)
    outs = jax.lax.optimization_barrier(outs)
    crops = []
    for l, r in enumerate(_RES):
        R = r + 2
        o3 = outs[l].reshape(_B, _COUT, R, R, R)
        o3 = o3[:, :, 1:R - 1, 1:R - 1, 1:R - 1]
        crops.append(o3.reshape(_B, _COUT, r ** 3))
    out = jnp.concatenate(crops, axis=2)
    return jnp.transpose(out, (0, 2, 1))


# speedup vs baseline: 1.4799x; 1.4799x over previous
import functools

import jax
import jax.numpy as jnp
from jax.experimental import pallas as pl

_RES = (16, 20, 24, 28, 32, 36, 40, 44)
_B, _CIN, _COUT = 2, 8, 8
_T = 4096


def _offsets():
    o = [0]
    for r in _RES:
        o.append(o[-1] + r ** 3)
    return tuple(o)


_OFF = _offsets()


def _round_up(x, m):
    return (x + m - 1) // m * m


def _lvl_body(x_ref, w_ref, b_ref, o_ref, *, r, h, T):
    t = pl.program_id(1)
    r2 = r * r
    p = t * T + jax.lax.broadcasted_iota(jnp.int32, (1, T), 1)
    cz = p % r
    cy = (p // r) % r
    cx = p // r2
    f32 = jnp.float32
    mx = ((cx >= 1).astype(f32), None, (cx <= r - 2).astype(f32))
    my = ((cy >= 1).astype(f32), None, (cy <= r - 2).astype(f32))
    mz = ((cz >= 1).astype(f32), None, (cz <= r - 2).astype(f32))
    pieces = []
    for dx in range(3):
        for dy in range(3):
            for dz in range(3):
                off = (dx - 1) * r2 + (dy - 1) * r + (dz - 1)
                start = h + off
                a = start // 128 * 128
                rem = start - a
                if rem == 0:
                    sl = x_ref[0, :, pl.ds(t * T + a, T)]
                else:
                    raw = x_ref[0, :, pl.ds(t * T + a, T + 128)]
                    sl = jax.lax.slice(raw, (0, rem), (_CIN, rem + T))
                m = None
                for mm in (mx[dx], my[dy], mz[dz]):
                    if mm is not None:
                        m = mm if m is None else m * mm
                if m is not None:
                    sl = sl * m
                pieces.append(sl)
    feats = jnp.concatenate(pieces, axis=0)
    acc = jnp.dot(w_ref[...], feats, preferred_element_type=f32)
    o_ref[0] = acc + b_ref[...]


def _lvl_conv(xp, wl, bl, *, r, s, h, nt, spad, blk, interpret=False):
    body = functools.partial(_lvl_body, r=r, h=h, T=_T)
    return pl.pallas_call(
        body,
        grid=(_B, nt),
        in_specs=[
            pl.BlockSpec((1, _CIN, spad), lambda b, t, j=blk: (b, 0, j)),
            pl.BlockSpec((_COUT, 27 * _CIN), lambda b, t: (0, 0)),
            pl.BlockSpec((_COUT, 1), lambda b, t: (0, 0)),
        ],
        out_specs=pl.BlockSpec((1, _COUT, _T), lambda b, t: (b, 0, t)),
        out_shape=jax.ShapeDtypeStruct((_B, _COUT, s), jnp.float32),
        interpret=interpret,
    )(xp, wl, bl)


def _level_params():
    params = []
    cur = 0
    end = 0
    for r in _RES:
        s = r ** 3
        h = r * r + r + 1
        nt = -(-s // _T)
        spad = (nt - 1) * _T + _round_up(_T + 2 * h, 128) + 128
        base = _round_up(cur, spad)
        params.append(dict(r=r, s=s, h=h, nt=nt, spad=spad, base=base))
        cur = base + h + s
        end = max(end, base + spad)
    total = _round_up(max(end, cur), 128)
    return params, total


_PARAMS, _P = _level_params()


def kernel(input, offsets, resolutions, weight, bias):
    xt = jnp.transpose(input, (0, 2, 1))
    pieces = []
    cur = 0
    for l, pr in enumerate(_PARAMS):
        target = pr["base"] + pr["h"]
        if target > cur:
            pieces.append(jnp.zeros((_B, _CIN, target - cur), jnp.float32))
        pieces.append(
            jax.lax.slice_in_dim(xt, _OFF[l], _OFF[l] + pr["s"], axis=2))
        cur = target + pr["s"]
    if _P > cur:
        pieces.append(jnp.zeros((_B, _CIN, _P - cur), jnp.float32))
    xcat = jnp.concatenate(pieces, axis=2)
    outs = []
    for l, pr in enumerate(_PARAMS):
        wl = jnp.transpose(weight[l].reshape(27 * _CIN, _COUT))
        bl = bias[l].reshape(_COUT, 1)
        outs.append(_lvl_conv(
            xcat, wl, bl, r=pr["r"], s=pr["s"], h=pr["h"], nt=pr["nt"],
            spad=pr["spad"], blk=pr["base"] // pr["spad"]))
    outs = jax.lax.optimization_barrier(outs)
    out = jnp.concatenate(outs, axis=2)
    return jnp.transpose(out, (0, 2, 1))


# speedup vs baseline: 1.9175x; 1.2957x over previous
import functools

import jax
import jax.numpy as jnp
from jax.experimental import pallas as pl

_RES = (16, 20, 24, 28, 32, 36, 40, 44)
_B, _CIN, _COUT = 2, 8, 8
_T = 4096


def _offsets():
    o = [0]
    for r in _RES:
        o.append(o[-1] + r ** 3)
    return tuple(o)


_OFF = _offsets()


def _round_up(x, m):
    return (x + m - 1) // m * m


def _lvl_body(x_ref, w_ref, b_ref, o_ref, *, r, h, T):
    t = pl.program_id(1)
    r2 = r * r
    p = t * T + jax.lax.broadcasted_iota(jnp.int32, (1, T), 1)
    cz = p % r
    cy = (p // r) % r
    cx = p // r2
    f32 = jnp.float32
    mx = ((cx >= 1).astype(f32), None, (cx <= r - 2).astype(f32))
    my = ((cy >= 1).astype(f32), None, (cy <= r - 2).astype(f32))
    mz = ((cz >= 1).astype(f32), None, (cz <= r - 2).astype(f32))
    pieces = []
    for dx in range(3):
        for dy in range(3):
            for dz in range(3):
                off = (dx - 1) * r2 + (dy - 1) * r + (dz - 1)
                start = h + off
                a = start // 128 * 128
                rem = start - a
                if rem == 0:
                    sl = x_ref[0, :, pl.ds(t * T + a, T)]
                else:
                    raw = x_ref[0, :, pl.ds(t * T + a, T + 128)]
                    sl = jax.lax.slice(raw, (0, rem), (_CIN, rem + T))
                m = None
                for mm in (mx[dx], my[dy], mz[dz]):
                    if mm is not None:
                        m = mm if m is None else m * mm
                if m is not None:
                    sl = sl * m
                pieces.append(sl)
    feats = jnp.concatenate(pieces, axis=0)
    acc = jnp.dot(w_ref[...], feats, preferred_element_type=f32)
    o_ref[0] = acc + b_ref[...]


def _lvl_conv(xp, wl, bl, *, r, s, h, nt, spad, interpret=False):
    body = functools.partial(_lvl_body, r=r, h=h, T=_T)
    return pl.pallas_call(
        body,
        grid=(_B, nt),
        in_specs=[
            pl.BlockSpec((1, _CIN, spad), lambda b, t: (b, 0, 0)),
            pl.BlockSpec((_COUT, 27 * _CIN), lambda b, t: (0, 0)),
            pl.BlockSpec((_COUT, 1), lambda b, t: (0, 0)),
        ],
        out_specs=pl.BlockSpec((1, _COUT, _T), lambda b, t: (b, 0, t)),
        out_shape=jax.ShapeDtypeStruct((_B, _COUT, s), jnp.float32),
        interpret=interpret,
    )(xp, wl, bl)


def kernel(input, offsets, resolutions, weight, bias):
    xt = jnp.transpose(input, (0, 2, 1))
    outs = []
    for l, r in enumerate(_RES):
        s = r ** 3
        h = r * r + r + 1
        nt = -(-s // _T)
        spad = (nt - 1) * _T + _round_up(_T + 2 * h, 128) + 128
        seg = jax.lax.slice_in_dim(xt, _OFF[l], _OFF[l] + s, axis=2)
        xp = jnp.pad(seg, ((0, 0), (0, 0), (h, spad - s - h)))
        wl = jnp.transpose(weight[l].reshape(27 * _CIN, _COUT))
        bl = bias[l].reshape(_COUT, 1)
        outs.append(_lvl_conv(xp, wl, bl, r=r, s=s, h=h, nt=nt, spad=spad))
    outs = jax.lax.optimization_barrier(outs)
    out = jnp.concatenate(outs, axis=2)
    return jnp.transpose(out, (0, 2, 1))
